# Initial kernel scaffold; baseline (speedup 1.0000x reference)
import jax
import jax.numpy as jnp
from jax.experimental import pallas as pl


def kernel(indices, codebook, W, b):
    raise NotImplementedError("write your pallas kernel here")


# speedup vs baseline: 1.5374x; 1.5374x over previous
import functools

import jax
import jax.numpy as jnp
from jax import lax
from jax.experimental import pallas as pl
from jax.experimental.pallas import tpu as pltpu
from jax.experimental.pallas import tpu_sc as plsc

CODE_DIM = 64
DIM = 256
IDX_CHUNK = 128


@functools.lru_cache(maxsize=None)
def _make_gather(ntok: int):
    info = plsc.get_sparse_core_info()
    nw = info.num_cores * info.num_subcores
    per_w = ntok // nw
    nchunk = per_w // IDX_CHUNK
    nc = info.num_cores
    mesh = plsc.VectorSubcoreMesh(core_axis_name="c", subcore_axis_name="s")

    @functools.partial(
        pl.kernel,
        mesh=mesh,
        compiler_params=pltpu.CompilerParams(use_tc_tiling_on_sc=False),
        out_type=jax.ShapeDtypeStruct((ntok, CODE_DIM), jnp.float32),
        scratch_types=[
            pltpu.VMEM((nchunk, IDX_CHUNK), jnp.int32),
            pltpu.VMEM((per_w, CODE_DIM), jnp.float32),
            pltpu.SemaphoreType.DMA,
        ],
    )
    def gather_kernel(idx_hbm, table_hbm, out_hbm, idx_v, rows_v, sem):
        wid = lax.axis_index("s") * nc + lax.axis_index("c")
        base = wid * per_w
        pltpu.sync_copy(idx_hbm.at[wid], idx_v)
        copies = []
        for j in range(nchunk):
            copies.append(
                pltpu.async_copy(
                    table_hbm.at[idx_v.at[j]],
                    rows_v.at[pl.ds(j * IDX_CHUNK, IDX_CHUNK)],
                    sem,
                )
            )
        for c in copies:
            c.wait()
        pltpu.sync_copy(rows_v, out_hbm.at[pl.ds(base, per_w)])

    def run(idx_flat, codebook):
        idx3 = idx_flat.reshape(nw, nchunk, IDX_CHUNK)
        return gather_kernel(idx3, codebook)

    return run


@functools.lru_cache(maxsize=None)
def _make_project(ntok: int):
    blk = 2048

    def mm(codes_ref, w_ref, b_ref, out_ref):
        out_ref[...] = (
            jnp.dot(codes_ref[...], w_ref[...], preferred_element_type=jnp.float32)
            + b_ref[...]
        )

    call = pl.pallas_call(
        mm,
        grid=(ntok // blk,),
        in_specs=[
            pl.BlockSpec((blk, CODE_DIM), lambda i: (i, 0)),
            pl.BlockSpec((CODE_DIM, DIM), lambda i: (0, 0)),
            pl.BlockSpec((1, DIM), lambda i: (0, 0)),
        ],
        out_specs=pl.BlockSpec((blk, DIM), lambda i: (i, 0)),
        out_shape=jax.ShapeDtypeStruct((ntok, DIM), jnp.float32),
    )

    def run(codes, W, bias):
        return call(codes, W, bias.reshape(1, DIM))

    return run


def kernel(indices, codebook, W, b):
    bsz, tlen = indices.shape
    ntok = bsz * tlen
    idx_flat = indices.reshape(ntok).astype(jnp.int32)
    codes = _make_gather(ntok)(idx_flat, codebook)
    out = _make_project(ntok)(codes, W, b)
    return out.reshape(bsz, tlen, DIM)


# speedup vs baseline: 1.6339x; 1.0628x over previous
import functools

import jax
import jax.numpy as jnp
from jax import lax
from jax.experimental import pallas as pl
from jax.experimental.pallas import tpu as pltpu
from jax.experimental.pallas import tpu_sc as plsc

CODE_DIM = 64
DIM = 256
IDX_CHUNK = 128


@functools.lru_cache(maxsize=None)
def _make_project(vocab: int):
    blk = 2048

    def mm(cb_ref, w_ref, b_ref, out_ref):
        out_ref[...] = (
            jnp.dot(cb_ref[...], w_ref[...], preferred_element_type=jnp.float32)
            + b_ref[...]
        )

    call = pl.pallas_call(
        mm,
        grid=(vocab // blk,),
        in_specs=[
            pl.BlockSpec((blk, CODE_DIM), lambda i: (i, 0)),
            pl.BlockSpec((CODE_DIM, DIM), lambda i: (0, 0)),
            pl.BlockSpec((1, DIM), lambda i: (0, 0)),
        ],
        out_specs=pl.BlockSpec((blk, DIM), lambda i: (i, 0)),
        out_shape=jax.ShapeDtypeStruct((vocab, DIM), jnp.float32),
    )

    def run(codebook, W, bias):
        return call(codebook, W, bias.reshape(1, DIM))

    return run


@functools.lru_cache(maxsize=None)
def _make_gather(ntok: int):
    info = plsc.get_sparse_core_info()
    nc = info.num_cores
    nw = nc * info.num_subcores
    per_w = ntok // nw
    nchunk = per_w // IDX_CHUNK
    mesh = plsc.VectorSubcoreMesh(core_axis_name="c", subcore_axis_name="s")

    @functools.partial(
        pl.kernel,
        mesh=mesh,
        out_type=jax.ShapeDtypeStruct((ntok, DIM), jnp.float32),
        scratch_types=[
            pltpu.VMEM((nchunk, IDX_CHUNK), jnp.int32),
            pltpu.VMEM((IDX_CHUNK, DIM), jnp.float32),
            pltpu.VMEM((IDX_CHUNK, DIM), jnp.float32),
            pltpu.SemaphoreType.DMA,
            pltpu.SemaphoreType.DMA,
            pltpu.SemaphoreType.DMA,
        ],
    )
    def gather_kernel(idx_hbm, proj_hbm, out_hbm, idx_v, buf0, buf1, gsem, ssem0, ssem1):
        wid = lax.axis_index("s") * nc + lax.axis_index("c")
        base = wid * per_w
        pltpu.sync_copy(idx_hbm.at[wid], idx_v)
        bufs = (buf0, buf1)
        ssems = (ssem0, ssem1)
        pending = [None, None]
        for j in range(nchunk):
            k = j % 2
            if pending[k] is not None:
                pending[k].wait()
            pltpu.async_copy(proj_hbm.at[idx_v.at[j]], bufs[k], gsem).wait()
            pending[k] = pltpu.async_copy(
                bufs[k], out_hbm.at[pl.ds(base + j * IDX_CHUNK, IDX_CHUNK)], ssems[k]
            )
        for p in pending:
            if p is not None:
                p.wait()

    def run(idx_flat, proj):
        idx3 = idx_flat.reshape(nw, nchunk, IDX_CHUNK)
        return gather_kernel(idx3, proj)

    return run


def kernel(indices, codebook, W, b):
    bsz, tlen = indices.shape
    ntok = bsz * tlen
    vocab = codebook.shape[0]
    idx_flat = indices.reshape(ntok).astype(jnp.int32)
    proj = _make_project(vocab)(codebook, W, b)
    out = _make_gather(ntok)(idx_flat, proj)
    return out.reshape(bsz, tlen, DIM)


# speedup vs baseline: 1.8811x; 1.1513x over previous
import functools

import jax
import jax.numpy as jnp
from jax import lax
from jax.experimental import pallas as pl
from jax.experimental.pallas import tpu as pltpu
from jax.experimental.pallas import tpu_sc as plsc

CODE_DIM = 64
DIM = 256
IDX_CHUNK = 128
NBUF = 3


@functools.lru_cache(maxsize=None)
def _make_project(vocab: int):
    blk = 2048

    def mm(cbt_ref, w_ref, b_ref, out_ref):
        out_ref[...] = (
            lax.dot_general(
                cbt_ref[...],
                w_ref[...],
                (((0,), (0,)), ((), ())),
                preferred_element_type=jnp.float32,
            )
            + b_ref[...]
        )

    call = pl.pallas_call(
        mm,
        grid=(vocab // blk,),
        in_specs=[
            pl.BlockSpec((CODE_DIM, blk), lambda i: (0, i)),
            pl.BlockSpec((CODE_DIM, DIM), lambda i: (0, 0)),
            pl.BlockSpec((1, DIM), lambda i: (0, 0)),
        ],
        out_specs=pl.BlockSpec((blk, DIM), lambda i: (i, 0)),
        out_shape=jax.ShapeDtypeStruct((vocab, DIM), jnp.float32),
    )

    def run(codebook, W, bias):
        return call(codebook.T, W, bias.reshape(1, DIM))

    return run


@functools.lru_cache(maxsize=None)
def _make_gather(bsz: int, tlen: int):
    ntok = bsz * tlen
    info = plsc.get_sparse_core_info()
    nc = info.num_cores
    nw = nc * info.num_subcores
    per_w = ntok // nw
    nchunk = per_w // IDX_CHUNK
    wper_row = tlen // per_w
    mesh = plsc.VectorSubcoreMesh(core_axis_name="c", subcore_axis_name="s")

    @functools.partial(
        pl.kernel,
        mesh=mesh,
        out_type=jax.ShapeDtypeStruct((ntok, DIM), jnp.float32),
        scratch_types=[
            pltpu.VMEM((per_w,), jnp.int32),
            [pltpu.VMEM((IDX_CHUNK, DIM), jnp.float32)] * NBUF,
            pltpu.SemaphoreType.DMA,
            [pltpu.SemaphoreType.DMA] * NBUF,
        ],
    )
    def gather_kernel(idx_hbm, proj_hbm, out_hbm, idx_v, bufs, gsem, ssems):
        wid = lax.axis_index("s") * nc + lax.axis_index("c")
        base = wid * per_w
        row = wid // wper_row
        col = (wid % wper_row) * per_w
        pltpu.sync_copy(idx_hbm.at[row, pl.ds(col, per_w)], idx_v)
        pending = [None] * NBUF
        for j in range(nchunk):
            k = j % NBUF
            if pending[k] is not None:
                pending[k].wait()
            pltpu.async_copy(
                proj_hbm.at[idx_v.at[pl.ds(j * IDX_CHUNK, IDX_CHUNK)]],
                bufs[k],
                gsem,
            ).wait()
            pending[k] = pltpu.async_copy(
                bufs[k], out_hbm.at[pl.ds(base + j * IDX_CHUNK, IDX_CHUNK)], ssems[k]
            )
        for p in pending:
            if p is not None:
                p.wait()

    return gather_kernel


def kernel(indices, codebook, W, b):
    bsz, tlen = indices.shape
    vocab = codebook.shape[0]
    proj = _make_project(vocab)(codebook, W, b)
    out = _make_gather(bsz, tlen)(indices.astype(jnp.int32), proj)
    return out.reshape(bsz, tlen, DIM)
